# baseline (speedup 1.0000x reference)
import jax
import jax.numpy as jnp
from jax.experimental import pallas as pl
from jax.experimental.pallas import tpu as pltpu

A_ = 5
C_ = 80
CIN = 768
CH = 1024
COUT = A_ * (5 + C_)
EPS = 1e-5
H = 32
W = 32
NPIX = H * W


def _body(x_ref, w1_ref, gm_ref, bt_ref, w2_ref, b2_ref, o_ref,
          xp_ref, acc_ref, y_ref, ssum_ref, ssq_ref, sc_ref, sh_ref):
    p = pl.program_id(0)
    b = pl.program_id(1)

    @pl.when(p == 0)
    def _conv1():
        @pl.when(b == 0)
        def _zero_pad():
            xp_ref[...] = jnp.zeros_like(xp_ref)

        xt = jnp.transpose(x_ref[0]).astype(jnp.bfloat16)
        xp_ref[1:H + 1, 1:W + 1, :] = xt.reshape(H, W, CIN)

        first = True
        for dx in range(3):
            xd = xp_ref[:, dx:dx + W, :].reshape((H + 2) * W, CIN)
            for dy in range(3):
                xm = xd[dy * W:dy * W + NPIX, :]
                d = jnp.dot(xm, w1_ref[dy, dx],
                            preferred_element_type=jnp.float32)
                if first:
                    acc_ref[...] = d
                    first = False
                else:
                    acc_ref[...] += d
        acc = acc_ref[...]
        ps = jnp.sum(acc, axis=0, keepdims=True)
        pq = jnp.sum(acc * acc, axis=0, keepdims=True)

        @pl.when(b == 0)
        def _init_stats():
            ssum_ref[...] = ps
            ssq_ref[...] = pq

        @pl.when(b != 0)
        def _acc_stats():
            ssum_ref[...] += ps
            ssq_ref[...] += pq

        y_ref[b] = acc.astype(jnp.bfloat16).reshape(H, W, CH)

    @pl.when(p == 1)
    def _conv2():
        @pl.when(b == 0)
        def _bn_affine():
            n = jnp.float32(pl.num_programs(1) * NPIX)
            mean = ssum_ref[...] / n
            var = ssq_ref[...] / n - mean * mean
            scale = gm_ref[...] * jax.lax.rsqrt(var + EPS)
            sc_ref[...] = scale
            sh_ref[...] = bt_ref[...] - mean * scale

        y = y_ref[b].reshape(NPIX, CH).astype(jnp.float32)
        z = y * sc_ref[...] + sh_ref[...]
        z = jnp.where(z > 0, z, 0.1 * z).astype(jnp.bfloat16)
        o = jnp.dot(z, w2_ref[...], preferred_element_type=jnp.float32)
        o_ref[0] = (o + b2_ref[...]).reshape(H, W, COUT)


def kernel(features, W1, gamma, beta, W2, b2):
    B = features.shape[0]
    xv = features.reshape(B, CIN, NPIX)
    w1 = jnp.transpose(W1, (2, 3, 1, 0)).astype(jnp.bfloat16)
    w2 = jnp.transpose(W2[:, :, 0, 0]).astype(jnp.bfloat16)

    out = pl.pallas_call(
        _body,
        grid=(2, B),
        in_specs=[
            pl.BlockSpec((1, CIN, NPIX), lambda p, b: ((1 - p) * b, 0, 0)),
            pl.BlockSpec((3, 3, CIN, CH), lambda p, b: (0, 0, 0, 0)),
            pl.BlockSpec((1, CH), lambda p, b: (0, 0)),
            pl.BlockSpec((1, CH), lambda p, b: (0, 0)),
            pl.BlockSpec((CH, COUT), lambda p, b: (0, 0)),
            pl.BlockSpec((1, COUT), lambda p, b: (0, 0)),
        ],
        out_specs=pl.BlockSpec((1, H, W, COUT), lambda p, b: (p * b, 0, 0, 0)),
        out_shape=jax.ShapeDtypeStruct((B, H, W, COUT), jnp.float32),
        scratch_shapes=[
            pltpu.VMEM((H + 2, W + 2, CIN), jnp.bfloat16),
            pltpu.VMEM((NPIX, CH), jnp.float32),
            pltpu.VMEM((B, H, W, CH), jnp.bfloat16),
            pltpu.VMEM((1, CH), jnp.float32),
            pltpu.VMEM((1, CH), jnp.float32),
            pltpu.VMEM((1, CH), jnp.float32),
            pltpu.VMEM((1, CH), jnp.float32),
        ],
        compiler_params=pltpu.CompilerParams(
            dimension_semantics=("arbitrary", "arbitrary")),
    )(xv, w1, gamma.reshape(1, CH), beta.reshape(1, CH), w2,
      b2.reshape(1, COUT))

    return out


# speedup vs baseline: 1.0623x; 1.0623x over previous
import jax
import jax.numpy as jnp
from jax.experimental import pallas as pl
from jax.experimental.pallas import tpu as pltpu

A_ = 5
C_ = 80
CIN = 768
CH = 1024
COUT = A_ * (5 + C_)
EPS = 1e-5
H = 32
W = 32
NPIX = H * W


def _body(x_ref, w1_ref, gm_ref, bt_ref, w2_ref, b2_ref, o_ref,
          acc_ref, y_ref, ssum_ref, ssq_ref, sc_ref, sh_ref):
    p = pl.program_id(0)
    b = pl.program_id(1)

    @pl.when(p == 0)
    def _conv1():
        x = x_ref[0]
        first = True
        for dx in range(3):
            xd = x[:, dx:dx + W, :].reshape((H + 2) * W, CIN)
            for dy in range(3):
                xm = xd[dy * W:dy * W + NPIX, :]
                d = jnp.dot(xm, w1_ref[dy, dx],
                            preferred_element_type=jnp.float32)
                if first:
                    acc_ref[...] = d
                    first = False
                else:
                    acc_ref[...] += d
        acc = acc_ref[...]
        ps = jnp.sum(acc, axis=0, keepdims=True)
        pq = jnp.sum(acc * acc, axis=0, keepdims=True)

        @pl.when(b == 0)
        def _init_stats():
            ssum_ref[...] = ps
            ssq_ref[...] = pq

        @pl.when(b != 0)
        def _acc_stats():
            ssum_ref[...] += ps
            ssq_ref[...] += pq

        y_ref[b] = acc.astype(jnp.bfloat16).reshape(H, W, CH)

    @pl.when(p == 1)
    def _conv2():
        @pl.when(b == 0)
        def _bn_affine():
            n = jnp.float32(pl.num_programs(1) * NPIX)
            mean = ssum_ref[...] / n
            var = ssq_ref[...] / n - mean * mean
            scale = gm_ref[...] * jax.lax.rsqrt(var + EPS)
            sc_ref[...] = scale
            sh_ref[...] = bt_ref[...] - mean * scale

        y = y_ref[b].reshape(NPIX, CH).astype(jnp.float32)
        z = y * sc_ref[...] + sh_ref[...]
        z = jnp.where(z > 0, z, 0.1 * z).astype(jnp.bfloat16)
        o = jnp.dot(z, w2_ref[...], preferred_element_type=jnp.float32)
        o_ref[0] = (o + b2_ref[...]).reshape(H, W, COUT)


def kernel(features, W1, gamma, beta, W2, b2):
    B = features.shape[0]
    xv = jnp.transpose(features, (0, 2, 3, 1))
    xv = jnp.pad(xv, ((0, 0), (1, 1), (1, 1), (0, 0))).astype(jnp.bfloat16)
    w1 = jnp.transpose(W1, (2, 3, 1, 0)).astype(jnp.bfloat16)
    w2 = jnp.transpose(W2[:, :, 0, 0]).astype(jnp.bfloat16)

    out = pl.pallas_call(
        _body,
        grid=(2, B),
        in_specs=[
            pl.BlockSpec((1, H + 2, W + 2, CIN),
                         lambda p, b: ((1 - p) * b, 0, 0, 0)),
            pl.BlockSpec((3, 3, CIN, CH), lambda p, b: (0, 0, 0, 0)),
            pl.BlockSpec((1, CH), lambda p, b: (0, 0)),
            pl.BlockSpec((1, CH), lambda p, b: (0, 0)),
            pl.BlockSpec((CH, COUT), lambda p, b: (0, 0)),
            pl.BlockSpec((1, COUT), lambda p, b: (0, 0)),
        ],
        out_specs=pl.BlockSpec((1, H, W, COUT), lambda p, b: (p * b, 0, 0, 0)),
        out_shape=jax.ShapeDtypeStruct((B, H, W, COUT), jnp.float32),
        scratch_shapes=[
            pltpu.VMEM((NPIX, CH), jnp.float32),
            pltpu.VMEM((B, H, W, CH), jnp.bfloat16),
            pltpu.VMEM((1, CH), jnp.float32),
            pltpu.VMEM((1, CH), jnp.float32),
            pltpu.VMEM((1, CH), jnp.float32),
            pltpu.VMEM((1, CH), jnp.float32),
        ],
        compiler_params=pltpu.CompilerParams(
            dimension_semantics=("arbitrary", "arbitrary")),
    )(xv, w1, gamma.reshape(1, CH), beta.reshape(1, CH), w2,
      b2.reshape(1, COUT))

    return out
